# Initial kernel scaffold; baseline (speedup 1.0000x reference)
import jax
import jax.numpy as jnp
from jax.experimental import pallas as pl


def kernel(s, t, edge_index):
    raise NotImplementedError("write your pallas kernel here")


# speedup vs baseline: 3.6901x; 3.6901x over previous
import functools

import jax
import jax.numpy as jnp
from jax import lax
from jax.experimental import pallas as pl
from jax.experimental.pallas import tpu as pltpu
from jax.experimental.pallas import tpu_sc as plsc

N_NODES = 10000
N_EDGES = 320000
D_FEAT = 128
NUM_CORES = 2
NUM_SUBCORES = 16
NUM_WORKERS = NUM_CORES * NUM_SUBCORES
EDGES_PER_WORKER = N_EDGES // NUM_WORKERS
CHUNK = 80
NUM_CHUNKS = EDGES_PER_WORKER // CHUNK
GROUPS = CHUNK // 16


def _decoder_body(s_hbm, t_hbm, si_hbm, di_hbm, out_hbm,
                  sidx, didx, srows, trows, outv, sem_s, sem_t):
    wid = lax.axis_index("s") * NUM_CORES + lax.axis_index("c")
    base = wid * EDGES_PER_WORKER
    pltpu.sync_copy(si_hbm.at[pl.ds(base, EDGES_PER_WORKER)], sidx)
    pltpu.sync_copy(di_hbm.at[pl.ds(base, EDGES_PER_WORKER)], didx)
    lanes = lax.iota(jnp.int32, 16)
    rot_idx = [(lanes + k) & 15 for k in (8, 4, 2, 1)]

    def chunk_body(ci, carry):
        off = pl.multiple_of(ci * CHUNK, 8)
        cp_s = pltpu.async_copy(s_hbm.at[sidx.at[pl.ds(off, CHUNK)]], srows, sem_s)
        cp_t = pltpu.async_copy(t_hbm.at[didx.at[pl.ds(off, CHUNK)]], trows, sem_t)
        cp_s.wait()
        cp_t.wait()

        def group_body(gi, carry2):
            e0 = gi * 16
            vec = jnp.zeros((16,), jnp.float32)
            for j in range(16):
                e = e0 + j
                acc = srows[e, pl.ds(0, 16)] * trows[e, pl.ds(0, 16)]
                for k in range(1, 8):
                    acc = acc + srows[e, pl.ds(k * 16, 16)] * trows[e, pl.ds(k * 16, 16)]
                for ri in rot_idx:
                    acc = acc + acc.at[ri].get(mode="promise_in_bounds")
                vec = jnp.where(lanes == j, acc, vec)
            outv[pl.ds(off + e0, 16)] = vec
            return carry2

        lax.fori_loop(0, GROUPS, group_body, 0)
        return carry

    lax.fori_loop(0, NUM_CHUNKS, chunk_body, 0)
    pltpu.sync_copy(outv, out_hbm.at[pl.ds(base, EDGES_PER_WORKER)])


@functools.partial(jax.jit)
def kernel(s, t, edge_index):
    ei = edge_index.astype(jnp.int32)
    mesh = plsc.VectorSubcoreMesh(core_axis_name="c", subcore_axis_name="s")
    run = pl.kernel(
        _decoder_body,
        out_type=jax.ShapeDtypeStruct((N_EDGES,), jnp.float32),
        mesh=mesh,
        scratch_types=[
            pltpu.VMEM((EDGES_PER_WORKER,), jnp.int32),
            pltpu.VMEM((EDGES_PER_WORKER,), jnp.int32),
            pltpu.VMEM((CHUNK, D_FEAT), jnp.float32),
            pltpu.VMEM((CHUNK, D_FEAT), jnp.float32),
            pltpu.VMEM((EDGES_PER_WORKER,), jnp.float32),
            pltpu.SemaphoreType.DMA,
            pltpu.SemaphoreType.DMA,
        ],
    )
    return run(s, t, ei[0], ei[1])


# speedup vs baseline: 5.1757x; 1.4026x over previous
import functools

import jax
import jax.numpy as jnp
from jax import lax
from jax.experimental import pallas as pl
from jax.experimental.pallas import tpu as pltpu
from jax.experimental.pallas import tpu_sc as plsc

N_NODES = 10000
N_EDGES = 320000
D_FEAT = 128
NUM_CORES = 2
NUM_SUBCORES = 16
NUM_WORKERS = NUM_CORES * NUM_SUBCORES
EDGES_PER_WORKER = N_EDGES // NUM_WORKERS
CHUNK = 80
NUM_CHUNKS = EDGES_PER_WORKER // CHUNK
GROUPS = CHUNK // 16


def _decoder_body(s_hbm, t_hbm, si_hbm, di_hbm, out_hbm,
                  sidx, didx, srows, trows, outv,
                  sem_s0, sem_s1, sem_t0, sem_t1):
    wid = lax.axis_index("s") * NUM_CORES + lax.axis_index("c")
    base = wid * EDGES_PER_WORKER
    pltpu.sync_copy(si_hbm.at[pl.ds(base, EDGES_PER_WORKER)], sidx)
    pltpu.sync_copy(di_hbm.at[pl.ds(base, EDGES_PER_WORKER)], didx)
    lanes = lax.iota(jnp.int32, 16)
    rot_idx = [(lanes + k) & 15 for k in (8, 4, 2, 1)]
    sem_s = (sem_s0, sem_s1)
    sem_t = (sem_t0, sem_t1)

    def gather_start(ci, b):
        off = pl.multiple_of(ci * CHUNK, 8)
        pltpu.async_copy(s_hbm.at[sidx.at[pl.ds(off, CHUNK)]], srows.at[b], sem_s[b])
        pltpu.async_copy(t_hbm.at[didx.at[pl.ds(off, CHUNK)]], trows.at[b], sem_t[b])

    def gather_wait(b):
        pltpu.make_async_copy(s_hbm.at[pl.ds(0, CHUNK)], srows.at[b], sem_s[b]).wait()
        pltpu.make_async_copy(t_hbm.at[pl.ds(0, CHUNK)], trows.at[b], sem_t[b]).wait()

    def compute(ci, b):
        off = ci * CHUNK

        def group_body(gi, carry2):
            e0 = gi * 16
            vec = jnp.zeros((16,), jnp.float32)
            for j in range(16):
                e = e0 + j
                acc = srows[b, e, pl.ds(0, 16)] * trows[b, e, pl.ds(0, 16)]
                for k in range(1, 8):
                    acc = acc + srows[b, e, pl.ds(k * 16, 16)] * trows[b, e, pl.ds(k * 16, 16)]
                for ri in rot_idx:
                    acc = acc + acc.at[ri].get(mode="promise_in_bounds")
                vec = jnp.where(lanes == j, acc, vec)
            outv[pl.ds(off + e0, 16)] = vec
            return carry2

        lax.fori_loop(0, GROUPS, group_body, 0)

    gather_start(0, 0)
    gather_start(1, 1)

    def pair_body(p, carry):
        ci0 = 2 * p
        for b in range(2):
            ci = ci0 + b
            gather_wait(b)
            compute(ci, b)

            @pl.when(ci + 2 < NUM_CHUNKS)
            def _():
                gather_start(ci + 2, b)
        return carry

    lax.fori_loop(0, NUM_CHUNKS // 2, pair_body, 0)
    gather_wait(0)
    compute(NUM_CHUNKS - 1, 0)
    pltpu.sync_copy(outv, out_hbm.at[pl.ds(base, EDGES_PER_WORKER)])


@functools.partial(jax.jit)
def kernel(s, t, edge_index):
    ei = edge_index.astype(jnp.int32)
    mesh = plsc.VectorSubcoreMesh(core_axis_name="c", subcore_axis_name="s")
    run = pl.kernel(
        _decoder_body,
        out_type=jax.ShapeDtypeStruct((N_EDGES,), jnp.float32),
        mesh=mesh,
        scratch_types=[
            pltpu.VMEM((EDGES_PER_WORKER,), jnp.int32),
            pltpu.VMEM((EDGES_PER_WORKER,), jnp.int32),
            pltpu.VMEM((2, CHUNK, D_FEAT), jnp.float32),
            pltpu.VMEM((2, CHUNK, D_FEAT), jnp.float32),
            pltpu.VMEM((EDGES_PER_WORKER,), jnp.float32),
            pltpu.SemaphoreType.DMA,
            pltpu.SemaphoreType.DMA,
            pltpu.SemaphoreType.DMA,
            pltpu.SemaphoreType.DMA,
        ],
    )
    return run(s, t, ei[0], ei[1])
